# Initial kernel scaffold; baseline (speedup 1.0000x reference)
import jax
import jax.numpy as jnp
from jax.experimental import pallas as pl


def kernel(Z, edges, y, W):
    raise NotImplementedError("write your pallas kernel here")


# speedup vs baseline: 34.4750x; 34.4750x over previous
import functools

import jax
import jax.numpy as jnp
from jax import lax
from jax.experimental import pallas as pl
from jax.experimental.pallas import tpu as pltpu
from jax.experimental.pallas import tpu_sc as plsc


def _uv_body(z_ref, w_ref, out_ref):
    W = w_ref[...]
    emb = z_ref.shape[1]
    Wl = W[0:emb, :]
    Wr = W[emb:2 * emb, :]
    Wm = jnp.concatenate([Wl, Wr], axis=1)
    P = lax.dot_general(Wm, z_ref[...], (((0,), (1,)), ((), ())),
                        preferred_element_type=jnp.float32)
    u = P[1:2, :] - P[0:1, :]
    v = P[3:4, :] - P[2:3, :]
    out_ref[...] = jnp.concatenate([u, v], axis=0)


def _node_tables(Z, W):
    n = Z.shape[0]
    return pl.pallas_call(
        _uv_body,
        out_shape=jax.ShapeDtypeStruct((2, n), jnp.float32),
    )(Z, W)


def _sc_loss_call(uv, src, dst, y):
    n = uv.shape[1]
    e = src.shape[0]
    info = plsc.get_sparse_core_info()
    nc, ns = info.num_cores, info.num_subcores
    nw = nc * ns
    epw = e // nw

    mesh = plsc.VectorSubcoreMesh(core_axis_name="c", subcore_axis_name="s")

    @functools.partial(
        pl.kernel,
        out_type=jax.ShapeDtypeStruct((nw, 16), jnp.float32),
        mesh=mesh,
        compiler_params=pltpu.CompilerParams(needs_layout_passes=False),
        scratch_types=[
            pltpu.VMEM((n,), jnp.float32),
            pltpu.VMEM((n,), jnp.float32),
            pltpu.VMEM((epw,), jnp.int32),
            pltpu.VMEM((epw,), jnp.int32),
            pltpu.VMEM((epw,), jnp.int32),
            pltpu.VMEM((16,), jnp.float32),
        ],
    )
    def sc_loss(uv_hbm, src_hbm, dst_hbm, y_hbm, out_hbm,
                u_v, v_v, s_v, d_v, y_v, acc_v):
        wid = lax.axis_index("s") * nc + lax.axis_index("c")
        base = wid * epw
        pltpu.sync_copy(uv_hbm.at[0], u_v)
        pltpu.sync_copy(uv_hbm.at[1], v_v)
        pltpu.sync_copy(src_hbm.at[pl.ds(base, epw)], s_v)
        pltpu.sync_copy(dst_hbm.at[pl.ds(base, epw)], d_v)
        pltpu.sync_copy(y_hbm.at[pl.ds(base, epw)], y_v)

        def body(i, acc):
            off = i * 16
            si = s_v[pl.ds(off, 16)]
            di = d_v[pl.ds(off, 16)]
            yv = y_v[pl.ds(off, 16)]
            d = plsc.load_gather(u_v, [si]) + plsc.load_gather(v_v, [di])
            x = (1.0 - 2.0 * yv.astype(jnp.float32)) * d
            t = jnp.exp(-jnp.abs(x))
            s = t / (2.0 + t)
            s2 = s * s
            p = s * (2.0 + s2 * (2.0 / 3.0 + s2 * (2.0 / 5.0
                     + s2 * (2.0 / 7.0 + s2 * (2.0 / 9.0)))))
            return acc + jnp.maximum(x, 0.0) + p

        acc = lax.fori_loop(0, epw // 16, body, jnp.zeros((16,), jnp.float32))
        acc_v[...] = acc
        pltpu.sync_copy(acc_v, out_hbm.at[wid])

    return sc_loss(uv, src, dst, y)


def kernel(Z, edges, y, W):
    uv = _node_tables(Z, W)
    src = edges[:, 0]
    dst = edges[:, 1]
    partials = _sc_loss_call(uv, src, dst, y)
    return jnp.sum(partials) / jnp.float32(edges.shape[0])
